# Initial kernel scaffold; baseline (speedup 1.0000x reference)
import jax
import jax.numpy as jnp
from jax.experimental import pallas as pl


def kernel(x, edge_index, batch, W1, as1, ad1, b1, W2, as2, ad2, b2, lw1, lb1, lw2, lb2, ow, ob):
    raise NotImplementedError("write your pallas kernel here")


# speedup vs baseline: 28.4548x; 28.4548x over previous
import functools

import jax
import jax.numpy as jnp
from jax import lax
from jax.experimental import pallas as pl
from jax.experimental.pallas import tpu as pltpu
from jax.experimental.pallas import tpu_sc as plsc

N = 10000
E = 320000
F = 128
C = 128
G = 16
OUT = 16
NEG = 0.2
EPS = 1e-16

NC = 2
NS = 16
NW = NC * NS
CHUNK = 128
NCHUNK = E // CHUNK
RPS = N // NS


def _embed_body(x_ref, w_ref, as_ref, ad_ref, h_ref, s_ref, d_ref):
    h = jnp.dot(x_ref[...], w_ref[...], preferred_element_type=jnp.float32)
    h_ref[...] = h
    s_ref[...] = jnp.sum(h * as_ref[...], axis=1, keepdims=True)
    d_ref[...] = jnp.sum(h * ad_ref[...], axis=1, keepdims=True)


def _embed(x, w, a_s, a_d):
    return pl.pallas_call(
        _embed_body,
        out_shape=[
            jax.ShapeDtypeStruct((N, C), jnp.float32),
            jax.ShapeDtypeStruct((N, 1), jnp.float32),
            jax.ShapeDtypeStruct((N, 1), jnp.float32),
        ],
    )(x, w, a_s, a_d)


def _mid_body(num_ref, den_ref, b_ref, w_ref, as_ref, ad_ref,
              h_ref, s_ref, d_ref):
    num = num_ref[0] + num_ref[1]
    den = den_ref[0] + den_ref[1]
    x = jnp.maximum(num / (den + EPS) + b_ref[...], 0.0)
    h = jnp.dot(x, w_ref[...], preferred_element_type=jnp.float32)
    h_ref[...] = h
    s_ref[...] = jnp.sum(h * as_ref[...], axis=1, keepdims=True)
    d_ref[...] = jnp.sum(h * ad_ref[...], axis=1, keepdims=True)


def _mid(num, den, b, w, a_s, a_d):
    return pl.pallas_call(
        _mid_body,
        out_shape=[
            jax.ShapeDtypeStruct((N, C), jnp.float32),
            jax.ShapeDtypeStruct((N, 1), jnp.float32),
            jax.ShapeDtypeStruct((N, 1), jnp.float32),
        ],
    )(num, den, b, w, a_s, a_d)


def _final_body(num_ref, den_ref, b_ref, batch_ref,
                lw1_ref, lb1_ref, lw2_ref, lb2_ref, ow_ref, ob_ref, out_ref):
    num = num_ref[0] + num_ref[1]
    den = den_ref[0] + den_ref[1]
    x = jnp.maximum(num / (den + EPS) + b_ref[...], 0.0)
    gid = lax.broadcasted_iota(jnp.int32, (G, N), 0)
    onehot = jnp.where(gid == batch_ref[...], 1.0, 0.0)
    sums = jnp.dot(onehot, x, preferred_element_type=jnp.float32)
    cnt = jnp.sum(onehot, axis=1, keepdims=True)
    g = sums / jnp.maximum(cnt, 1.0)
    g = jnp.maximum(jnp.dot(g, lw1_ref[...],
                            preferred_element_type=jnp.float32) + lb1_ref[...], 0.0)
    g = jnp.maximum(jnp.dot(g, lw2_ref[...],
                            preferred_element_type=jnp.float32) + lb2_ref[...], 0.0)
    out_ref[...] = jnp.dot(g, ow_ref[...],
                           preferred_element_type=jnp.float32) + ob_ref[...]


def _final(num, den, b, batch, lw1, lb1, lw2, lb2, ow, ob):
    return pl.pallas_call(
        _final_body,
        out_shape=jax.ShapeDtypeStruct((G, OUT), jnp.float32),
    )(num, den, b, batch, lw1, lb1, lw2, lb2, ow, ob)


def _sc_edge(h, a_s, a_d, src, dst):
    mesh = plsc.VectorSubcoreMesh(core_axis_name="c", subcore_axis_name="s")

    @functools.partial(
        pl.kernel,
        out_type=[
            jax.ShapeDtypeStruct((NC, N, C), jnp.float32),
            jax.ShapeDtypeStruct((NC * N,), jnp.float32),
        ],
        mesh=mesh,
        scratch_types=[
            pltpu.VMEM_SHARED((N, C), jnp.float32),
            pltpu.VMEM_SHARED((N,), jnp.float32),
            pltpu.VMEM((CHUNK, C), jnp.float32),
            pltpu.VMEM((CHUNK,), jnp.int32),
            pltpu.VMEM((CHUNK,), jnp.int32),
            pltpu.VMEM((CHUNK,), jnp.float32),
            pltpu.VMEM((CHUNK,), jnp.float32),
            pltpu.VMEM((CHUNK,), jnp.float32),
            pltpu.VMEM((1024,), jnp.float32),
            pltpu.SemaphoreType.DMA,
            pltpu.SemaphoreType.DMA,
            pltpu.SemaphoreType.DMA,
        ],
    )
    def k(h_hbm, as_hbm, ad_hbm, src_hbm, dst_hbm, num_out, den_out,
          num_sh, den_sh, rows_v, src_v, dst_v, as_v, ad_v, p_v, den_st,
          sem_a, sem_b, sem_r):
        c = lax.axis_index("c")
        s = lax.axis_index("s")
        wid = s * NC + c

        zero16 = jnp.zeros((16,), jnp.float32)

        def zero_row(i, carry):
            for j in range(C // 16):
                rows_v[i, pl.ds(16 * j, 16)] = zero16
            return carry
        lax.fori_loop(0, CHUNK, zero_row, 0)

        def zero_st(i, carry):
            den_st[pl.ds(i * 16, 16)] = zero16
            return carry
        lax.fori_loop(0, 64, zero_st, 0)

        @pl.when(s < 15)
        def _():
            for i in range(6):
                r0 = pl.multiple_of(s * 624 + i * 104, 8)
                pltpu.sync_copy(rows_v.at[pl.ds(0, 104)],
                                num_sh.at[pl.ds(r0, 104)])

        @pl.when(s == 15)
        def _():
            for i in range(5):
                r0 = pl.multiple_of(9360 + i * 128, 8)
                pltpu.sync_copy(rows_v.at[pl.ds(0, 128)],
                                num_sh.at[pl.ds(r0, 128)])

        @pl.when(s < 10)
        def _():
            pltpu.sync_copy(den_st.at[pl.ds(0, 1000)],
                            den_sh.at[pl.ds(s * 1000, 1000)])

        plsc.subcore_barrier()

        lo = wid * NCHUNK // NW
        hi = (wid + 1) * NCHUNK // NW

        def body(ci, carry):
            base = ci * CHUNK
            pltpu.sync_copy(src_hbm.at[pl.ds(base, CHUNK)], src_v)
            pltpu.sync_copy(dst_hbm.at[pl.ds(base, CHUNK)], dst_v)
            cp_r = pltpu.async_copy(h_hbm.at[src_v], rows_v, sem_r)
            cp_a = pltpu.async_copy(as_hbm.at[src_v], as_v, sem_a)
            cp_b = pltpu.async_copy(ad_hbm.at[dst_v], ad_v, sem_b)
            cp_a.wait()
            cp_b.wait()
            for j in range(CHUNK // 16):
                e = as_v[pl.ds(16 * j, 16)] + ad_v[pl.ds(16 * j, 16)]
                e = jnp.where(e >= 0.0, e, NEG * e)
                p_v[pl.ds(16 * j, 16)] = jnp.exp(e)
            pltpu.sync_copy(p_v, den_sh.at[dst_v], add=True)
            cp_r.wait()

            def scale(g2, carry2):
                pvec = p_v[pl.ds(16 * g2, 16)]
                for k2 in range(16):
                    pb = jnp.full((16,), pvec[k2], jnp.float32)
                    row = 16 * g2 + k2
                    for j in range(C // 16):
                        rows_v[row, pl.ds(16 * j, 16)] = (
                            rows_v[row, pl.ds(16 * j, 16)] * pb)
                return carry2
            lax.fori_loop(0, CHUNK // 16, scale, 0)
            pltpu.sync_copy(rows_v, num_sh.at[dst_v], add=True)
            return carry
        lax.fori_loop(lo, hi, body, 0)

        plsc.subcore_barrier()

        @pl.when(s < 15)
        def _():
            for i in range(6):
                r0 = pl.multiple_of(s * 624 + i * 104, 8)
                pltpu.sync_copy(num_sh.at[pl.ds(r0, 104)],
                                rows_v.at[pl.ds(0, 104)])
                pltpu.sync_copy(rows_v.at[pl.ds(0, 104)],
                                num_out.at[c, pl.ds(r0, 104)])

        @pl.when(s == 15)
        def _():
            for i in range(5):
                r0 = pl.multiple_of(9360 + i * 128, 8)
                pltpu.sync_copy(num_sh.at[pl.ds(r0, 128)],
                                rows_v.at[pl.ds(0, 128)])
                pltpu.sync_copy(rows_v.at[pl.ds(0, 128)],
                                num_out.at[c, pl.ds(r0, 128)])

        @pl.when(s < 10)
        def _():
            d0 = pl.multiple_of(c * N + s * 1000, 8)
            pltpu.sync_copy(den_sh.at[pl.ds(s * 1000, 1000)],
                            den_st.at[pl.ds(0, 1000)])
            pltpu.sync_copy(den_st.at[pl.ds(0, 1000)],
                            den_out.at[pl.ds(d0, 1000)])

    return k(h, a_s, a_d, src, dst)


def kernel(x, edge_index, batch, W1, as1, ad1, b1, W2, as2, ad2, b2,
           lw1, lb1, lw2, lb2, ow, ob):
    src = edge_index[0]
    dst = edge_index[1]
    as1r = as1.reshape(1, C)
    ad1r = ad1.reshape(1, C)
    as2r = as2.reshape(1, C)
    ad2r = ad2.reshape(1, C)

    h1, s1, d1 = _embed(x, W1, as1r, ad1r)
    num1, den1 = _sc_edge(h1, s1.reshape(N), d1.reshape(N), src, dst)
    h2, s2, d2 = _mid(num1, den1.reshape(NC, N, 1), b1.reshape(1, C),
                      W2, as2r, ad2r)
    num2, den2 = _sc_edge(h2, s2.reshape(N), d2.reshape(N), src, dst)
    return _final(num2, den2.reshape(NC, N, 1), b2.reshape(1, C),

                  batch.reshape(1, N), lw1, lb1.reshape(1, -1),
                  lw2, lb2.reshape(1, -1), ow, ob.reshape(1, -1))


# speedup vs baseline: 55.0403x; 1.9343x over previous
import functools

import jax
import jax.numpy as jnp
from jax import lax
from jax.experimental import pallas as pl
from jax.experimental.pallas import tpu as pltpu
from jax.experimental.pallas import tpu_sc as plsc

N = 10000
E = 320000
F = 128
C = 128
G = 16
OUT = 16
NEG = 0.2
EPS = 1e-16

NC = 2
NS = 16
NW = NC * NS
CHUNK = 80
NCHUNK = E // CHUNK
CPW = NCHUNK // NW
NB = 4


def _embed_body(x_ref, w_ref, as_ref, ad_ref, h_ref, s_ref, d_ref):
    h = jnp.dot(x_ref[...], w_ref[...], preferred_element_type=jnp.float32)
    h_ref[...] = h
    s_ref[...] = jnp.sum(h * as_ref[...], axis=1, keepdims=True)
    d_ref[...] = jnp.sum(h * ad_ref[...], axis=1, keepdims=True)


def _embed(x, w, a_s, a_d):
    return pl.pallas_call(
        _embed_body,
        out_shape=[
            jax.ShapeDtypeStruct((N, C), jnp.float32),
            jax.ShapeDtypeStruct((N, 1), jnp.float32),
            jax.ShapeDtypeStruct((N, 1), jnp.float32),
        ],
    )(x, w, a_s, a_d)


def _mid_body(num_ref, den_ref, b_ref, w_ref, as_ref, ad_ref,
              h_ref, s_ref, d_ref):
    num = num_ref[0] + num_ref[1]
    den = den_ref[0] + den_ref[1]
    x = jnp.maximum(num / (den + EPS) + b_ref[...], 0.0)
    h = jnp.dot(x, w_ref[...], preferred_element_type=jnp.float32)
    h_ref[...] = h
    s_ref[...] = jnp.sum(h * as_ref[...], axis=1, keepdims=True)
    d_ref[...] = jnp.sum(h * ad_ref[...], axis=1, keepdims=True)


def _mid(num, den, b, w, a_s, a_d):
    return pl.pallas_call(
        _mid_body,
        out_shape=[
            jax.ShapeDtypeStruct((N, C), jnp.float32),
            jax.ShapeDtypeStruct((N, 1), jnp.float32),
            jax.ShapeDtypeStruct((N, 1), jnp.float32),
        ],
    )(num, den, b, w, a_s, a_d)


def _final_body(num_ref, den_ref, b_ref, batch_ref,
                lw1_ref, lb1_ref, lw2_ref, lb2_ref, ow_ref, ob_ref, out_ref):
    num = num_ref[0] + num_ref[1]
    den = den_ref[0] + den_ref[1]
    x = jnp.maximum(num / (den + EPS) + b_ref[...], 0.0)
    gid = lax.broadcasted_iota(jnp.int32, (G, N), 0)
    onehot = jnp.where(gid == batch_ref[...], 1.0, 0.0)
    sums = jnp.dot(onehot, x, preferred_element_type=jnp.float32)
    cnt = jnp.sum(onehot, axis=1, keepdims=True)
    g = sums / jnp.maximum(cnt, 1.0)
    g = jnp.maximum(jnp.dot(g, lw1_ref[...],
                            preferred_element_type=jnp.float32) + lb1_ref[...], 0.0)
    g = jnp.maximum(jnp.dot(g, lw2_ref[...],
                            preferred_element_type=jnp.float32) + lb2_ref[...], 0.0)
    out_ref[...] = jnp.dot(g, ow_ref[...],
                           preferred_element_type=jnp.float32) + ob_ref[...]


def _final(num, den, b, batch, lw1, lb1, lw2, lb2, ow, ob):
    return pl.pallas_call(
        _final_body,
        out_shape=jax.ShapeDtypeStruct((G, OUT), jnp.float32),
    )(num, den, b, batch, lw1, lb1, lw2, lb2, ow, ob)


def _sc_edge(h, a_s, a_d, src, dst):
    mesh = plsc.VectorSubcoreMesh(core_axis_name="c", subcore_axis_name="s")

    @functools.partial(
        pl.kernel,
        out_type=[
            jax.ShapeDtypeStruct((NC, N, C), jnp.float32),
            jax.ShapeDtypeStruct((NC * N,), jnp.float32),
        ],
        mesh=mesh,
        scratch_types=[
            pltpu.VMEM_SHARED((N, C), jnp.float32),
            pltpu.VMEM_SHARED((N,), jnp.float32),
            [pltpu.VMEM((CHUNK, C), jnp.float32)] * NB,
            [pltpu.VMEM((CHUNK,), jnp.int32)] * NB,
            [pltpu.VMEM((CHUNK,), jnp.int32)] * NB,
            [pltpu.VMEM((CHUNK,), jnp.float32)] * NB,
            [pltpu.VMEM((CHUNK,), jnp.float32)] * NB,
            [pltpu.VMEM((CHUNK,), jnp.float32)] * NB,
            pltpu.VMEM((1024,), jnp.float32),
            [pltpu.SemaphoreType.DMA] * NB,
            [pltpu.SemaphoreType.DMA] * NB,
            [pltpu.SemaphoreType.DMA] * NB,
            [pltpu.SemaphoreType.DMA] * NB,
            [pltpu.SemaphoreType.DMA] * NB,
            [pltpu.SemaphoreType.DMA] * NB,
            [pltpu.SemaphoreType.DMA] * NB,
        ],
    )
    def k(h_hbm, as_hbm, ad_hbm, src_hbm, dst_hbm, num_out, den_out,
          num_sh, den_sh, rows, srcc, dstc, asb, adb, pb, den_st,
          si_s, si_d, sg_as, sg_ad, sg_r, ss_p, ss_r):
        c = lax.axis_index("c")
        s = lax.axis_index("s")
        wid = s * NC + c
        base_ch = wid * CPW

        zero16 = jnp.zeros((16,), jnp.float32)

        def zero_row(i, carry):
            for j in range(C // 16):
                rows[0][i, pl.ds(16 * j, 16)] = zero16
            return carry
        lax.fori_loop(0, CHUNK, zero_row, 0)

        def zero_st(i, carry):
            den_st[pl.ds(i * 16, 16)] = zero16
            return carry
        lax.fori_loop(0, 64, zero_st, 0)

        @pl.when(s < 15)
        def _():
            for i in range(7):
                r0 = pl.multiple_of(s * 624 + i * 80, 8)
                pltpu.sync_copy(rows[0].at[pl.ds(0, 80)],
                                num_sh.at[pl.ds(r0, 80)])
            r1 = pl.multiple_of(s * 624 + 560, 8)
            pltpu.sync_copy(rows[0].at[pl.ds(0, 64)],
                            num_sh.at[pl.ds(r1, 64)])

        @pl.when(s == 15)
        def _():
            for i in range(8):
                r0 = pl.multiple_of(9360 + i * 80, 8)
                pltpu.sync_copy(rows[0].at[pl.ds(0, 80)],
                                num_sh.at[pl.ds(r0, 80)])

        @pl.when(s < 10)
        def _():
            pltpu.sync_copy(den_st.at[pl.ds(0, 1000)],
                            den_sh.at[pl.ds(s * 1000, 1000)])

        plsc.subcore_barrier()

        def wait_scat(b):
            pltpu.make_async_copy(pb[b], den_sh.at[dstc[b]], ss_p[b]).wait()
            pltpu.make_async_copy(rows[b], num_sh.at[dstc[b]], ss_r[b]).wait()

        def issue_idx(ci, b, first):
            if not first:
                wait_scat(b)
            e0 = (base_ch + ci) * CHUNK
            pltpu.async_copy(src_hbm.at[pl.ds(e0, CHUNK)], srcc[b], si_s[b])
            pltpu.async_copy(dst_hbm.at[pl.ds(e0, CHUNK)], dstc[b], si_d[b])

        def issue_gath(ci, b):
            e0 = (base_ch + ci) * CHUNK
            pltpu.make_async_copy(src_hbm.at[pl.ds(e0, CHUNK)], srcc[b],
                                  si_s[b]).wait()
            pltpu.make_async_copy(dst_hbm.at[pl.ds(e0, CHUNK)], dstc[b],
                                  si_d[b]).wait()
            pltpu.async_copy(h_hbm.at[srcc[b]], rows[b], sg_r[b])
            pltpu.async_copy(as_hbm.at[srcc[b]], asb[b], sg_as[b])
            pltpu.async_copy(ad_hbm.at[dstc[b]], adb[b], sg_ad[b])

        def proc(b):
            pltpu.make_async_copy(as_hbm.at[srcc[b]], asb[b], sg_as[b]).wait()
            pltpu.make_async_copy(ad_hbm.at[dstc[b]], adb[b], sg_ad[b]).wait()
            for j in range(CHUNK // 16):
                e = asb[b][pl.ds(16 * j, 16)] + adb[b][pl.ds(16 * j, 16)]
                e = jnp.where(e >= 0.0, e, NEG * e)
                pb[b][pl.ds(16 * j, 16)] = jnp.exp(e)
            pltpu.async_copy(pb[b], den_sh.at[dstc[b]], ss_p[b], add=True)
            pltpu.make_async_copy(h_hbm.at[srcc[b]], rows[b], sg_r[b]).wait()

            def scale(g2, carry2):
                pvec = pb[b][pl.ds(16 * g2, 16)]
                for k2 in range(16):
                    pbc = jnp.full((16,), pvec[k2], jnp.float32)
                    row = 16 * g2 + k2
                    for j in range(C // 16):
                        rows[b][row, pl.ds(16 * j, 16)] = (
                            rows[b][row, pl.ds(16 * j, 16)] * pbc)
                return carry2
            lax.fori_loop(0, CHUNK // 16, scale, 0)
            pltpu.async_copy(rows[b], num_sh.at[dstc[b]], ss_r[b], add=True)

        issue_idx(0, 0, True)
        issue_idx(1, 1, True)
        issue_gath(0, 0)
        issue_idx(2, 2, True)
        issue_gath(1, 1)
        proc(0)
        issue_idx(3, 3, True)
        issue_gath(2, 2)
        proc(1)
        issue_idx(4, 0, False)
        issue_gath(3, 3)
        proc(2)
        issue_idx(5, 1, False)
        issue_gath(4, 0)
        proc(3)

        def body(g, carry):
            issue_idx(4 * g + 2, 2, False)
            issue_gath(4 * g + 1, 1)
            proc(0)
            issue_idx(4 * g + 3, 3, False)
            issue_gath(4 * g + 2, 2)
            proc(1)
            issue_idx(4 * g + 4, 0, False)
            issue_gath(4 * g + 3, 3)
            proc(2)
            issue_idx(4 * g + 5, 1, False)
            issue_gath(4 * g + 4, 0)
            proc(3)
            return carry
        lax.fori_loop(1, (CPW - 5) // 4, body, 0)

        issue_idx(122, 2, False)
        issue_gath(121, 1)
        proc(0)
        issue_idx(123, 3, False)
        issue_gath(122, 2)
        proc(1)
        issue_idx(124, 0, False)
        issue_gath(123, 3)
        proc(2)
        issue_gath(124, 0)
        proc(3)
        proc(0)

        wait_scat(1)
        wait_scat(2)
        wait_scat(3)
        wait_scat(0)

        plsc.subcore_barrier()

        @pl.when(s < 15)
        def _():
            for i in range(7):
                r0 = pl.multiple_of(s * 624 + i * 80, 8)
                pltpu.sync_copy(num_sh.at[pl.ds(r0, 80)],
                                rows[0].at[pl.ds(0, 80)])
                pltpu.sync_copy(rows[0].at[pl.ds(0, 80)],
                                num_out.at[c, pl.ds(r0, 80)])
            r1 = pl.multiple_of(s * 624 + 560, 8)
            pltpu.sync_copy(num_sh.at[pl.ds(r1, 64)],
                            rows[0].at[pl.ds(0, 64)])
            pltpu.sync_copy(rows[0].at[pl.ds(0, 64)],
                            num_out.at[c, pl.ds(r1, 64)])

        @pl.when(s == 15)
        def _():
            for i in range(8):
                r0 = pl.multiple_of(9360 + i * 80, 8)
                pltpu.sync_copy(num_sh.at[pl.ds(r0, 80)],
                                rows[0].at[pl.ds(0, 80)])
                pltpu.sync_copy(rows[0].at[pl.ds(0, 80)],
                                num_out.at[c, pl.ds(r0, 80)])

        @pl.when(s < 10)
        def _():
            d0 = pl.multiple_of(c * N + s * 1000, 8)
            pltpu.sync_copy(den_sh.at[pl.ds(s * 1000, 1000)],
                            den_st.at[pl.ds(0, 1000)])
            pltpu.sync_copy(den_st.at[pl.ds(0, 1000)],
                            den_out.at[pl.ds(d0, 1000)])

    return k(h, a_s, a_d, src, dst)


def kernel(x, edge_index, batch, W1, as1, ad1, b1, W2, as2, ad2, b2,
           lw1, lb1, lw2, lb2, ow, ob):
    src = edge_index[0]
    dst = edge_index[1]
    as1r = as1.reshape(1, C)
    ad1r = ad1.reshape(1, C)
    as2r = as2.reshape(1, C)
    ad2r = ad2.reshape(1, C)

    h1, s1, d1 = _embed(x, W1, as1r, ad1r)
    num1, den1 = _sc_edge(h1, s1.reshape(N), d1.reshape(N), src, dst)
    h2, s2, d2 = _mid(num1, den1.reshape(NC, N, 1), b1.reshape(1, C),
                      W2, as2r, ad2r)
    num2, den2 = _sc_edge(h2, s2.reshape(N), d2.reshape(N), src, dst)
    return _final(num2, den2.reshape(NC, N, 1), b2.reshape(1, C),

                  batch.reshape(1, N), lw1, lb1.reshape(1, -1),
                  lw2, lb2.reshape(1, -1), ow, ob.reshape(1, -1))


# speedup vs baseline: 61.8927x; 1.1245x over previous
import functools

import jax
import jax.numpy as jnp
from jax import lax
from jax.experimental import pallas as pl
from jax.experimental.pallas import tpu as pltpu
from jax.experimental.pallas import tpu_sc as plsc

N = 10000
E = 320000
F = 128
C = 128
G = 16
OUT = 16
NEG = 0.2
EPS = 1e-16

NC = 2
NS = 16
NW = NC * NS
CHUNK = 80
NCHUNK = E // CHUNK
CPW = NCHUNK // NW
NB = 4


def _embed_body(x_ref, w_ref, as_ref, ad_ref, h_ref, s_ref, d_ref):
    h = jnp.dot(x_ref[...], w_ref[...], preferred_element_type=jnp.float32)
    h_ref[...] = h
    s_ref[...] = jnp.sum(h * as_ref[...], axis=1)
    d_ref[...] = jnp.sum(h * ad_ref[...], axis=1)


def _embed(x, w, a_s, a_d):
    return pl.pallas_call(
        _embed_body,
        out_shape=[
            jax.ShapeDtypeStruct((N, C), jnp.float32),
            jax.ShapeDtypeStruct((N,), jnp.float32),
            jax.ShapeDtypeStruct((N,), jnp.float32),
        ],
    )(x, w, a_s, a_d)


def _mid_body(num_ref, den_ref, b_ref, w_ref, as_ref, ad_ref,
              h_ref, s_ref, d_ref):
    num = num_ref[0] + num_ref[1]
    den = (den_ref[pl.ds(0, N)] + den_ref[pl.ds(N, N)]).reshape(N, 1)
    x = jnp.maximum(num / (den + EPS) + b_ref[...], 0.0)
    h = jnp.dot(x, w_ref[...], preferred_element_type=jnp.float32)
    h_ref[...] = h
    s_ref[...] = jnp.sum(h * as_ref[...], axis=1)
    d_ref[...] = jnp.sum(h * ad_ref[...], axis=1)


def _mid(num, den, b, w, a_s, a_d):
    return pl.pallas_call(
        _mid_body,
        out_shape=[
            jax.ShapeDtypeStruct((N, C), jnp.float32),
            jax.ShapeDtypeStruct((N,), jnp.float32),
            jax.ShapeDtypeStruct((N,), jnp.float32),
        ],
    )(num, den, b, w, a_s, a_d)


def _final_body(num_ref, den_ref, b_ref, batch_ref,
                lw1_ref, lb1_ref, lw2_ref, lb2_ref, ow_ref, ob_ref, out_ref):
    num = num_ref[0] + num_ref[1]
    den = (den_ref[pl.ds(0, N)] + den_ref[pl.ds(N, N)]).reshape(N, 1)
    x = jnp.maximum(num / (den + EPS) + b_ref[...], 0.0)
    gid = lax.broadcasted_iota(jnp.int32, (G, N), 0)
    onehot = jnp.where(gid == batch_ref[...], 1.0, 0.0)
    sums = jnp.dot(onehot, x, preferred_element_type=jnp.float32)
    cnt = jnp.sum(onehot, axis=1, keepdims=True)
    g = sums / jnp.maximum(cnt, 1.0)
    g = jnp.maximum(jnp.dot(g, lw1_ref[...],
                            preferred_element_type=jnp.float32) + lb1_ref[...], 0.0)
    g = jnp.maximum(jnp.dot(g, lw2_ref[...],
                            preferred_element_type=jnp.float32) + lb2_ref[...], 0.0)
    out_ref[...] = jnp.dot(g, ow_ref[...],
                           preferred_element_type=jnp.float32) + ob_ref[...]


def _final(num, den, b, batch, lw1, lb1, lw2, lb2, ow, ob):
    return pl.pallas_call(
        _final_body,
        out_shape=jax.ShapeDtypeStruct((G, OUT), jnp.float32),
    )(num, den, b, batch, lw1, lb1, lw2, lb2, ow, ob)


def _sc_edge(h, a_s, a_d, src, dst):
    mesh = plsc.VectorSubcoreMesh(core_axis_name="c", subcore_axis_name="s")

    @functools.partial(
        pl.kernel,
        out_type=[
            jax.ShapeDtypeStruct((NC, N, C), jnp.float32),
            jax.ShapeDtypeStruct((NC * N,), jnp.float32),
        ],
        mesh=mesh,
        scratch_types=[
            pltpu.VMEM_SHARED((N, C), jnp.float32),
            pltpu.VMEM_SHARED((N,), jnp.float32),
            [pltpu.VMEM((CHUNK, C), jnp.float32)] * NB,
            [pltpu.VMEM((CHUNK,), jnp.int32)] * NB,
            [pltpu.VMEM((CHUNK,), jnp.int32)] * NB,
            [pltpu.VMEM((CHUNK,), jnp.float32)] * NB,
            [pltpu.VMEM((CHUNK,), jnp.float32)] * NB,
            [pltpu.VMEM((CHUNK,), jnp.float32)] * NB,
            pltpu.VMEM((1024,), jnp.float32),
            [pltpu.SemaphoreType.DMA] * NB,
            [pltpu.SemaphoreType.DMA] * NB,
            [pltpu.SemaphoreType.DMA] * NB,
            [pltpu.SemaphoreType.DMA] * NB,
            [pltpu.SemaphoreType.DMA] * NB,
            [pltpu.SemaphoreType.DMA] * NB,
            [pltpu.SemaphoreType.DMA] * NB,
        ],
    )
    def k(h_hbm, as_hbm, ad_hbm, src_hbm, dst_hbm, num_out, den_out,
          num_sh, den_sh, rows, srcc, dstc, asb, adb, pb, den_st,
          si_s, si_d, sg_as, sg_ad, sg_r, ss_p, ss_r):
        c = lax.axis_index("c")
        s = lax.axis_index("s")
        wid = s * NC + c
        base_ch = wid * CPW

        zero16 = jnp.zeros((16,), jnp.float32)

        def zero_row(i, carry):
            for j in range(C // 16):
                rows[0][i, pl.ds(16 * j, 16)] = zero16
            return carry
        lax.fori_loop(0, CHUNK, zero_row, 0)

        def zero_st(i, carry):
            den_st[pl.ds(i * 16, 16)] = zero16
            return carry
        lax.fori_loop(0, 64, zero_st, 0)

        @pl.when(s < 15)
        def _():
            for i in range(7):
                r0 = pl.multiple_of(s * 624 + i * 80, 8)
                pltpu.sync_copy(rows[0].at[pl.ds(0, 80)],
                                num_sh.at[pl.ds(r0, 80)])
            r1 = pl.multiple_of(s * 624 + 560, 8)
            pltpu.sync_copy(rows[0].at[pl.ds(0, 64)],
                            num_sh.at[pl.ds(r1, 64)])

        @pl.when(s == 15)
        def _():
            for i in range(8):
                r0 = pl.multiple_of(9360 + i * 80, 8)
                pltpu.sync_copy(rows[0].at[pl.ds(0, 80)],
                                num_sh.at[pl.ds(r0, 80)])

        @pl.when(s < 10)
        def _():
            pltpu.sync_copy(den_st.at[pl.ds(0, 1000)],
                            den_sh.at[pl.ds(s * 1000, 1000)])

        plsc.subcore_barrier()

        def wait_scat(b):
            pltpu.make_async_copy(pb[b], den_sh.at[dstc[b]], ss_p[b]).wait()
            pltpu.make_async_copy(rows[b], num_sh.at[dstc[b]], ss_r[b]).wait()

        def issue_idx(ci, b, first):
            if not first:
                wait_scat(b)
            e0 = (base_ch + ci) * CHUNK
            pltpu.async_copy(src_hbm.at[pl.ds(e0, CHUNK)], srcc[b], si_s[b])
            pltpu.async_copy(dst_hbm.at[pl.ds(e0, CHUNK)], dstc[b], si_d[b])

        def issue_gath(ci, b):
            e0 = (base_ch + ci) * CHUNK
            pltpu.make_async_copy(src_hbm.at[pl.ds(e0, CHUNK)], srcc[b],
                                  si_s[b]).wait()
            pltpu.make_async_copy(dst_hbm.at[pl.ds(e0, CHUNK)], dstc[b],
                                  si_d[b]).wait()
            pltpu.async_copy(h_hbm.at[srcc[b]], rows[b], sg_r[b])
            pltpu.async_copy(as_hbm.at[srcc[b]], asb[b], sg_as[b])
            pltpu.async_copy(ad_hbm.at[dstc[b]], adb[b], sg_ad[b])

        def proc(b):
            pltpu.make_async_copy(as_hbm.at[srcc[b]], asb[b], sg_as[b]).wait()
            pltpu.make_async_copy(ad_hbm.at[dstc[b]], adb[b], sg_ad[b]).wait()
            for j in range(CHUNK // 16):
                e = asb[b][pl.ds(16 * j, 16)] + adb[b][pl.ds(16 * j, 16)]
                e = jnp.where(e >= 0.0, e, NEG * e)
                pb[b][pl.ds(16 * j, 16)] = jnp.exp(e)
            pltpu.async_copy(pb[b], den_sh.at[dstc[b]], ss_p[b], add=True)
            pltpu.make_async_copy(h_hbm.at[srcc[b]], rows[b], sg_r[b]).wait()

            def scale(g2, carry2):
                pvec = pb[b][pl.ds(16 * g2, 16)]
                for k2 in range(16):
                    pbc = jnp.full((16,), pvec[k2], jnp.float32)
                    row = 16 * g2 + k2
                    for j in range(C // 16):
                        rows[b][row, pl.ds(16 * j, 16)] = (
                            rows[b][row, pl.ds(16 * j, 16)] * pbc)
                return carry2
            lax.fori_loop(0, CHUNK // 16, scale, 0)
            pltpu.async_copy(rows[b], num_sh.at[dstc[b]], ss_r[b], add=True)

        issue_idx(0, 0, True)
        issue_idx(1, 1, True)
        issue_gath(0, 0)
        issue_idx(2, 2, True)
        issue_gath(1, 1)
        proc(0)
        issue_idx(3, 3, True)
        issue_gath(2, 2)
        proc(1)
        issue_idx(4, 0, False)
        issue_gath(3, 3)
        proc(2)
        issue_idx(5, 1, False)
        issue_gath(4, 0)
        proc(3)

        def body(g, carry):
            issue_idx(4 * g + 2, 2, False)
            issue_gath(4 * g + 1, 1)
            proc(0)
            issue_idx(4 * g + 3, 3, False)
            issue_gath(4 * g + 2, 2)
            proc(1)
            issue_idx(4 * g + 4, 0, False)
            issue_gath(4 * g + 3, 3)
            proc(2)
            issue_idx(4 * g + 5, 1, False)
            issue_gath(4 * g + 4, 0)
            proc(3)
            return carry
        lax.fori_loop(1, (CPW - 5) // 4, body, 0)

        issue_idx(122, 2, False)
        issue_gath(121, 1)
        proc(0)
        issue_idx(123, 3, False)
        issue_gath(122, 2)
        proc(1)
        issue_idx(124, 0, False)
        issue_gath(123, 3)
        proc(2)
        issue_gath(124, 0)
        proc(3)
        proc(0)

        wait_scat(1)
        wait_scat(2)
        wait_scat(3)
        wait_scat(0)

        plsc.subcore_barrier()

        @pl.when(s < 15)
        def _():
            for i in range(7):
                r0 = pl.multiple_of(s * 624 + i * 80, 8)
                pltpu.sync_copy(num_sh.at[pl.ds(r0, 80)],
                                rows[0].at[pl.ds(0, 80)])
                pltpu.sync_copy(rows[0].at[pl.ds(0, 80)],
                                num_out.at[c, pl.ds(r0, 80)])
            r1 = pl.multiple_of(s * 624 + 560, 8)
            pltpu.sync_copy(num_sh.at[pl.ds(r1, 64)],
                            rows[0].at[pl.ds(0, 64)])
            pltpu.sync_copy(rows[0].at[pl.ds(0, 64)],
                            num_out.at[c, pl.ds(r1, 64)])

        @pl.when(s == 15)
        def _():
            for i in range(8):
                r0 = pl.multiple_of(9360 + i * 80, 8)
                pltpu.sync_copy(num_sh.at[pl.ds(r0, 80)],
                                rows[0].at[pl.ds(0, 80)])
                pltpu.sync_copy(rows[0].at[pl.ds(0, 80)],
                                num_out.at[c, pl.ds(r0, 80)])

        @pl.when(s < 10)
        def _():
            d0 = pl.multiple_of(c * N + s * 1000, 8)
            pltpu.sync_copy(den_sh.at[pl.ds(s * 1000, 1000)],
                            den_st.at[pl.ds(0, 1000)])
            pltpu.sync_copy(den_st.at[pl.ds(0, 1000)],
                            den_out.at[pl.ds(d0, 1000)])

    return k(h, a_s, a_d, src, dst)


def kernel(x, edge_index, batch, W1, as1, ad1, b1, W2, as2, ad2, b2,
           lw1, lb1, lw2, lb2, ow, ob):
    src = edge_index[0]
    dst = edge_index[1]
    as1r = as1.reshape(1, C)
    ad1r = ad1.reshape(1, C)
    as2r = as2.reshape(1, C)
    ad2r = ad2.reshape(1, C)

    h1, s1, d1 = _embed(x, W1, as1r, ad1r)
    num1, den1 = _sc_edge(h1, s1, d1, src, dst)
    h2, s2, d2 = _mid(num1, den1, b1.reshape(1, C), W2, as2r, ad2r)
    num2, den2 = _sc_edge(h2, s2, d2, src, dst)
    return _final(num2, den2, b2.reshape(1, C),
                  batch.reshape(1, N), lw1, lb1.reshape(1, -1),
                  lw2, lb2.reshape(1, -1), ow, ob.reshape(1, -1))
